# baseline (device time: 6692 ns/iter reference)
import jax
import jax.numpy as jnp
from jax import lax
from jax.experimental import pallas as pl
from jax.experimental.pallas import tpu as pltpu

EPS = 1e-5
Y_SIZE = 2


def kernel(x, gamma):
    m, n = x.shape
    n_global = Y_SIZE * n
    blocks = m // 128

    def body(x_ref, g_ref, out_ref, partial_ref, recv_ref, send_sem, recv_sem):
        my_x = lax.axis_index("x")
        my_y = lax.axis_index("y")
        nbr = (my_x, 1 - my_y)

        barrier_sem = pltpu.get_barrier_semaphore()
        pl.semaphore_signal(
            barrier_sem, inc=1, device_id=nbr,
            device_id_type=pl.DeviceIdType.MESH,
        )

        ident = jnp.asarray(
            lax.broadcasted_iota(jnp.int32, (128, 128), 0)
            == lax.broadcasted_iota(jnp.int32, (128, 128), 1),
            jnp.float32,
        )

        xv = x_ref[:, :]
        col = jnp.sum(xv * xv, axis=1, keepdims=True)
        for i in range(blocks):
            cb = col[i * 128:(i + 1) * 128, :]
            partial_ref[i:i + 1, :] = lax.dot_general(
                cb, ident, (((0,), (0,)), ((), ())),
                preferred_element_type=jnp.float32,
            )

        pl.semaphore_wait(barrier_sem, 1)

        rdma = pltpu.make_async_remote_copy(
            src_ref=partial_ref,
            dst_ref=recv_ref,
            send_sem=send_sem,
            recv_sem=recv_sem,
            device_id=nbr,
            device_id_type=pl.DeviceIdType.MESH,
        )
        rdma.start()

        out_ref[:, :] = g_ref[:, :] * xv

        rdma.wait()

        total = partial_ref[:, :] + recv_ref[:, :]
        inv = lax.rsqrt(total / n_global + EPS)
        cols = [
            lax.dot_general(
                ident, inv[i:i + 1, :], (((1,), (1,)), ((), ())),
                preferred_element_type=jnp.float32,
            )
            for i in range(blocks)
        ]
        inv_col = jnp.concatenate(cols, axis=0)
        out_ref[:, :] = out_ref[:, :] * inv_col

    return pl.pallas_call(
        body,
        out_shape=jax.ShapeDtypeStruct((m, n), jnp.float32),
        in_specs=[
            pl.BlockSpec(memory_space=pltpu.VMEM),
            pl.BlockSpec(memory_space=pltpu.VMEM),
        ],
        out_specs=pl.BlockSpec(memory_space=pltpu.VMEM),
        scratch_shapes=[
            pltpu.VMEM((m // 128, 128), jnp.float32),
            pltpu.VMEM((m // 128, 128), jnp.float32),
            pltpu.SemaphoreType.DMA,
            pltpu.SemaphoreType.DMA,
        ],
        compiler_params=pltpu.CompilerParams(collective_id=0),
    )(x, gamma.reshape(1, n))


# device time: 6551 ns/iter; 1.0215x vs baseline; 1.0215x over previous
import jax
import jax.numpy as jnp
from jax import lax
from jax.experimental import pallas as pl
from jax.experimental.pallas import tpu as pltpu

EPS = 1e-5
Y_SIZE = 2


def kernel(x, gamma):
    m, n = x.shape
    n_global = Y_SIZE * n
    blocks = m // 128

    def body(x_ref, g_ref, out_ref, partial_ref, recv_ref, send_sem, recv_sem):
        my_x = lax.axis_index("x")
        my_y = lax.axis_index("y")
        nbr = (my_x, 1 - my_y)

        barrier_sem = pltpu.get_barrier_semaphore()
        pl.semaphore_signal(
            barrier_sem, inc=1, device_id=nbr,
            device_id_type=pl.DeviceIdType.MESH,
        )

        ident = jnp.asarray(
            lax.broadcasted_iota(jnp.int32, (128, 128), 0)
            == lax.broadcasted_iota(jnp.int32, (128, 128), 1),
            jnp.float32,
        )

        xv = x_ref[:, :]
        x3 = xv.reshape(blocks, 128, n)
        partial_ref[:, :] = jnp.sum(x3 * x3, axis=2)

        pl.semaphore_wait(barrier_sem, 1)

        rdma = pltpu.make_async_remote_copy(
            src_ref=partial_ref,
            dst_ref=recv_ref,
            send_sem=send_sem,
            recv_sem=recv_sem,
            device_id=nbr,
            device_id_type=pl.DeviceIdType.MESH,
        )
        rdma.start()

        out_ref[:, :] = g_ref[:, :] * xv

        rdma.wait()

        total = partial_ref[:, :] + recv_ref[:, :]
        inv = lax.rsqrt(total / n_global + EPS)
        cols = [
            lax.dot_general(
                ident, inv[i:i + 1, :], (((1,), (1,)), ((), ())),
                preferred_element_type=jnp.float32,
            )
            for i in range(blocks)
        ]
        inv_col = jnp.concatenate(cols, axis=0)
        out_ref[:, :] = out_ref[:, :] * inv_col

    return pl.pallas_call(
        body,
        out_shape=jax.ShapeDtypeStruct((m, n), jnp.float32),
        in_specs=[
            pl.BlockSpec(memory_space=pltpu.VMEM),
            pl.BlockSpec(memory_space=pltpu.VMEM),
        ],
        out_specs=pl.BlockSpec(memory_space=pltpu.VMEM),
        scratch_shapes=[
            pltpu.VMEM((m // 128, 128), jnp.float32),
            pltpu.VMEM((m // 128, 128), jnp.float32),
            pltpu.SemaphoreType.DMA,
            pltpu.SemaphoreType.DMA,
        ],
        compiler_params=pltpu.CompilerParams(collective_id=0),
    )(x, gamma.reshape(1, n))


# device time: 6454 ns/iter; 1.0369x vs baseline; 1.0150x over previous
import jax
import jax.numpy as jnp
from jax import lax
from jax.experimental import pallas as pl
from jax.experimental.pallas import tpu as pltpu

EPS = 1e-5
Y_SIZE = 2


def kernel(x, gamma):
    m, n = x.shape
    n_global = Y_SIZE * n
    blocks = m // 128

    def body(x_ref, g_ref, out_ref, partial_ref, recv_ref, send_sem, recv_sem):
        my_x = lax.axis_index("x")
        my_y = lax.axis_index("y")
        nbr = (my_x, 1 - my_y)

        barrier_sem = pltpu.get_barrier_semaphore()
        pl.semaphore_signal(
            barrier_sem, inc=1, device_id=nbr,
            device_id_type=pl.DeviceIdType.MESH,
        )

        xv = x_ref[:, :]
        x3 = xv.reshape(blocks, 128, n)
        partial_ref[:, :] = jnp.sum(x3 * x3, axis=2)

        pl.semaphore_wait(barrier_sem, 1)

        rdma = pltpu.make_async_remote_copy(
            src_ref=partial_ref,
            dst_ref=recv_ref,
            send_sem=send_sem,
            recv_sem=recv_sem,
            device_id=nbr,
            device_id_type=pl.DeviceIdType.MESH,
        )
        rdma.start()

        out_ref[:, :] = g_ref[:, :] * xv

        rdma.wait()

        total = partial_ref[:, :] + recv_ref[:, :]
        inv = lax.rsqrt(total / n_global + EPS)
        out3 = out_ref[:, :].reshape(blocks, 128, n) * inv[:, :, None]
        out_ref[:, :] = out3.reshape(m, n)

    return pl.pallas_call(
        body,
        out_shape=jax.ShapeDtypeStruct((m, n), jnp.float32),
        in_specs=[
            pl.BlockSpec(memory_space=pltpu.VMEM),
            pl.BlockSpec(memory_space=pltpu.VMEM),
        ],
        out_specs=pl.BlockSpec(memory_space=pltpu.VMEM),
        scratch_shapes=[
            pltpu.VMEM((m // 128, 128), jnp.float32),
            pltpu.VMEM((m // 128, 128), jnp.float32),
            pltpu.SemaphoreType.DMA,
            pltpu.SemaphoreType.DMA,
        ],
        compiler_params=pltpu.CompilerParams(collective_id=0),
    )(x, gamma.reshape(1, n))
